# baseline (speedup 1.0000x reference)
import functools

import jax
import jax.numpy as jnp
from jax import lax
from jax.experimental import pallas as pl
from jax.experimental.pallas import tpu as pltpu
from jax.experimental.pallas import tpu_sc as plsc

N = 10000
E = 320000
D = 128
K = 8

NC = 2
NS = 16
NW = NC * NS
C = 128
NCHUNK = 79
EP = NW * NCHUNK * C
NP = 10240
RPT = NP // NS
WB = 128
NWB = RPT // WB
DEGW = 16

def _onescat(idx_hbm, ones_hbm, zrows_hbm, p_hbm, cidx0, cidx1, ones_v, zbuf,
             ssem0, ssem1, acc):
    cid = lax.axis_index("c")
    sid = lax.axis_index("s")
    wid = sid * NC + cid

    pltpu.sync_copy(ones_hbm, ones_v)
    pltpu.sync_copy(zrows_hbm, zbuf)
    for b in range(NWB):
        pltpu.sync_copy(zbuf, acc.at[pl.ds(sid * RPT + b * WB, WB)])
    plsc.subcore_barrier()

    cidx = (cidx0, cidx1)
    ssem = (ssem0, ssem1)

    pltpu.sync_copy(idx_hbm.at[wid].at[0], cidx0)
    pltpu.async_copy(ones_v, acc.at[cidx0], ssem0, add=True)

    def step(jj, cur):
        @pl.when(jj >= 2)
        def _():
            pltpu.make_async_copy(ones_v, acc.at[cidx[cur]], ssem[cur]).wait()

        pltpu.sync_copy(idx_hbm.at[wid].at[jj], cidx[cur])
        pltpu.async_copy(ones_v, acc.at[cidx[cur]], ssem[cur], add=True)

    def body(jj, _):
        @pl.when(jj % 2 == 0)
        def _():
            step(jj, 0)

        @pl.when(jj % 2 == 1)
        def _():
            step(jj, 1)

        return 0

    lax.fori_loop(1, NCHUNK, body, 0)
    pltpu.make_async_copy(ones_v, acc.at[cidx0], ssem0).wait()
    pltpu.make_async_copy(ones_v, acc.at[cidx1], ssem1).wait()
    plsc.subcore_barrier()

    for b in range(NWB):
        row0 = sid * RPT + b * WB
        pltpu.sync_copy(acc.at[pl.ds(row0, WB)], p_hbm.at[cid].at[pl.ds(row0, WB)])


def _hop(u_hbm, src_hbm, dst_hbm, zrows_hbm, p_hbm,
         sidx0, sidx1, didx0, didx1, rows0, rows1, gsem0, gsem1,
         ssem0, ssem1, acc):
    cid = lax.axis_index("c")
    sid = lax.axis_index("s")
    wid = sid * NC + cid

    pltpu.sync_copy(zrows_hbm, rows0)
    for b in range(NWB):
        pltpu.sync_copy(rows0, acc.at[pl.ds(sid * RPT + b * WB, WB)])
    plsc.subcore_barrier()

    sidx = (sidx0, sidx1)
    didx = (didx0, didx1)
    rows = (rows0, rows1)
    gsem = (gsem0, gsem1)
    ssem = (ssem0, ssem1)

    pltpu.sync_copy(src_hbm.at[wid].at[0], sidx0)
    pltpu.sync_copy(dst_hbm.at[wid].at[0], didx0)
    pltpu.async_copy(u_hbm.at[sidx0], rows0, gsem0)

    def step(jj, cur, nxt):
        pltpu.make_async_copy(u_hbm.at[sidx[cur]], rows[cur], gsem[cur]).wait()
        pltpu.async_copy(rows[cur], acc.at[didx[cur]], ssem[cur], add=True)

        @pl.when(jj + 1 < NCHUNK)
        def _():
            pltpu.sync_copy(src_hbm.at[wid].at[jj + 1], sidx[nxt])
            pltpu.sync_copy(dst_hbm.at[wid].at[jj + 1], didx[nxt])
            pltpu.async_copy(u_hbm.at[sidx[nxt]], rows[nxt], gsem[nxt])

        pltpu.make_async_copy(rows[cur], acc.at[didx[cur]], ssem[cur]).wait()

    def body(jj, _):
        @pl.when(jj % 2 == 0)
        def _():
            step(jj, 0, 1)

        @pl.when(jj % 2 == 1)
        def _():
            step(jj, 1, 0)

        return 0

    lax.fori_loop(0, NCHUNK, body, 0)
    plsc.subcore_barrier()

    for b in range(NWB):
        row0 = sid * RPT + b * WB
        pltpu.sync_copy(acc.at[pl.ds(row0, WB)], p_hbm.at[cid].at[pl.ds(row0, WB)])


@functools.lru_cache(maxsize=None)
def _sc_kernels():
    mesh = plsc.VectorSubcoreMesh(
        core_axis_name="c", subcore_axis_name="s", num_cores=NC, num_subcores=NS
    )
    deg = pl.kernel(
        _onescat,
        out_type=jax.ShapeDtypeStruct((NC, NP, D), jnp.float32),
        mesh=mesh,
        scratch_types=[
            pltpu.VMEM((C,), jnp.int32),
            pltpu.VMEM((C,), jnp.int32),
            pltpu.VMEM((C, D), jnp.float32),
            pltpu.VMEM((WB, D), jnp.float32),
            pltpu.SemaphoreType.DMA,
            pltpu.SemaphoreType.DMA,
            pltpu.VMEM_SHARED((NP, D), jnp.float32),
        ],
    )
    hop = pl.kernel(
        _hop,
        out_type=jax.ShapeDtypeStruct((NC, NP, D), jnp.float32),
        mesh=mesh,
        scratch_types=[
            pltpu.VMEM((C,), jnp.int32),
            pltpu.VMEM((C,), jnp.int32),
            pltpu.VMEM((C,), jnp.int32),
            pltpu.VMEM((C,), jnp.int32),
            pltpu.VMEM((WB, D), jnp.float32),
            pltpu.VMEM((WB, D), jnp.float32),
            pltpu.SemaphoreType.DMA,
            pltpu.SemaphoreType.DMA,
            pltpu.SemaphoreType.DMA,
            pltpu.SemaphoreType.DMA,
            pltpu.VMEM_SHARED((NP, D), jnp.float32),
        ],
    )
    return deg, hop


_BR = 2048


def _relu_scale(x, s):
    def body(x_ref, s_ref, o_ref):
        o_ref[...] = jnp.maximum(x_ref[...], 0.0) * s_ref[...]

    return pl.pallas_call(
        body,
        grid=(NP // _BR,),
        in_specs=[
            pl.BlockSpec((_BR, D), lambda i: (i, 0)),
            pl.BlockSpec((_BR, 1), lambda i: (i, 0)),
        ],
        out_specs=pl.BlockSpec((_BR, D), lambda i: (i, 0)),
        out_shape=jax.ShapeDtypeStruct((NP, D), jnp.float32),
    )(x, s)


def _combine(p0, p1, s):
    def body(a_ref, b_ref, s_ref, o_ref):
        o_ref[...] = (a_ref[...] + b_ref[...]) * s_ref[...]

    return pl.pallas_call(
        body,
        grid=(NP // _BR,),
        in_specs=[
            pl.BlockSpec((_BR, D), lambda i: (i, 0)),
            pl.BlockSpec((_BR, D), lambda i: (i, 0)),
            pl.BlockSpec((_BR, 1), lambda i: (i, 0)),
        ],
        out_specs=pl.BlockSpec((_BR, D), lambda i: (i, 0)),
        out_shape=jax.ShapeDtypeStruct((NP, D), jnp.float32),
    )(p0, p1, s)


def kernel(x, edge_index):
    pad = jnp.full((EP - E,), NP - 1, dtype=jnp.int32)
    src = jnp.concatenate([edge_index[0], pad]).reshape(NW, NCHUNK, C)
    dst = jnp.concatenate([edge_index[1], pad]).reshape(NW, NCHUNK, C)

    deg_call, hop_call = _sc_kernels()
    ones_rows = jnp.ones((C, D), jnp.float32)
    zrows = jnp.zeros((WB, D), jnp.float32)
    ps = deg_call(src, ones_rows, zrows)
    pd = deg_call(dst, ones_rows, zrows)
    deg_s = ps[0, :, 0] + ps[1, :, 0]
    deg_d = pd[0, :, 0] + pd[1, :, 0]
    s_src = jnp.where(deg_s > 0, lax.rsqrt(jnp.maximum(deg_s, 1.0)), 0.0)
    s_dst = jnp.where(deg_d > 0, lax.rsqrt(jnp.maximum(deg_d, 1.0)), 0.0)

    xp = jnp.pad(x, ((0, NP - N), (0, 0)))
    u = _relu_scale(xp, s_src[:, None])
    s_mid = (s_src * s_dst)[:, None]
    for t in range(K):
        p = hop_call(u, src, dst, zrows)
        u = _combine(p[0], p[1], s_mid if t < K - 1 else s_dst[:, None])
    return u[:N]


# speedup vs baseline: 1.1696x; 1.1696x over previous
import functools

import jax
import jax.numpy as jnp
from jax import lax
from jax.experimental import pallas as pl
from jax.experimental.pallas import tpu as pltpu
from jax.experimental.pallas import tpu_sc as plsc

N = 10000
E = 320000
D = 128
K = 8

NC = 2
NS = 16
NW = NC * NS
C = 128
NCHUNK = 79
EP = NW * NCHUNK * C
NP = 10240
RPT = NP // NS
WB = 128
NWB = RPT // WB
DEGW = 16

def _onescat(idx_hbm, ones_hbm, zrows_hbm, p_hbm, cidx0, cidx1, ones_v, zbuf,
             ssem0, ssem1, acc):
    cid = lax.axis_index("c")
    sid = lax.axis_index("s")
    wid = sid * NC + cid

    pltpu.sync_copy(ones_hbm, ones_v)
    pltpu.sync_copy(zrows_hbm, zbuf)
    for b in range(NWB):
        pltpu.sync_copy(zbuf, acc.at[pl.ds(sid * RPT + b * WB, WB)])
    plsc.subcore_barrier()

    cidx = (cidx0, cidx1)
    ssem = (ssem0, ssem1)

    pltpu.sync_copy(idx_hbm.at[wid].at[0], cidx0)
    pltpu.async_copy(ones_v, acc.at[cidx0], ssem0, add=True)

    def step(jj, cur):
        @pl.when(jj >= 2)
        def _():
            pltpu.make_async_copy(ones_v, acc.at[cidx[cur]], ssem[cur]).wait()

        pltpu.sync_copy(idx_hbm.at[wid].at[jj], cidx[cur])
        pltpu.async_copy(ones_v, acc.at[cidx[cur]], ssem[cur], add=True)

    def body(jj, _):
        @pl.when(jj % 2 == 0)
        def _():
            step(jj, 0)

        @pl.when(jj % 2 == 1)
        def _():
            step(jj, 1)

        return 0

    lax.fori_loop(1, NCHUNK, body, 0)
    pltpu.make_async_copy(ones_v, acc.at[cidx0], ssem0).wait()
    pltpu.make_async_copy(ones_v, acc.at[cidx1], ssem1).wait()
    plsc.subcore_barrier()

    for b in range(NWB):
        row0 = sid * RPT + b * WB
        pltpu.sync_copy(acc.at[pl.ds(row0, WB)], p_hbm.at[cid].at[pl.ds(row0, WB)])


def _hop(u_hbm, src_hbm, dst_hbm, zrows_hbm, p_hbm,
         sidx0, sidx1, didx0, didx1, rows0, rows1, gsem0, gsem1,
         ssem0, ssem1, acc):
    cid = lax.axis_index("c")
    sid = lax.axis_index("s")
    wid = sid * NC + cid

    pltpu.sync_copy(zrows_hbm, rows0)
    for b in range(NWB):
        pltpu.sync_copy(rows0, acc.at[pl.ds(sid * RPT + b * WB, WB)])
    plsc.subcore_barrier()

    sidx = (sidx0, sidx1)
    didx = (didx0, didx1)
    rows = (rows0, rows1)
    gsem = (gsem0, gsem1)
    ssem = (ssem0, ssem1)

    pltpu.sync_copy(src_hbm.at[wid].at[0], sidx0)
    pltpu.sync_copy(dst_hbm.at[wid].at[0], didx0)
    pltpu.async_copy(u_hbm.at[sidx0], rows0, gsem0)

    def step(jj, cur, nxt):
        @pl.when(jj + 1 < NCHUNK)
        def _():
            pltpu.sync_copy(src_hbm.at[wid].at[jj + 1], sidx[nxt])
            pltpu.sync_copy(dst_hbm.at[wid].at[jj + 1], didx[nxt])
            pltpu.async_copy(u_hbm.at[sidx[nxt]], rows[nxt], gsem[nxt])

        pltpu.make_async_copy(u_hbm.at[sidx[cur]], rows[cur], gsem[cur]).wait()
        pltpu.sync_copy(rows[cur], acc.at[didx[cur]], add=True)

    def body(jj, _):
        @pl.when(jj % 2 == 0)
        def _():
            step(jj, 0, 1)

        @pl.when(jj % 2 == 1)
        def _():
            step(jj, 1, 0)

        return 0

    lax.fori_loop(0, NCHUNK, body, 0)
    plsc.subcore_barrier()

    for b in range(NWB):
        row0 = sid * RPT + b * WB
        pltpu.sync_copy(acc.at[pl.ds(row0, WB)], p_hbm.at[cid].at[pl.ds(row0, WB)])


@functools.lru_cache(maxsize=None)
def _sc_kernels():
    mesh = plsc.VectorSubcoreMesh(
        core_axis_name="c", subcore_axis_name="s", num_cores=NC, num_subcores=NS
    )
    deg = pl.kernel(
        _onescat,
        out_type=jax.ShapeDtypeStruct((NC, NP, D), jnp.float32),
        mesh=mesh,
        scratch_types=[
            pltpu.VMEM((C,), jnp.int32),
            pltpu.VMEM((C,), jnp.int32),
            pltpu.VMEM((C, D), jnp.float32),
            pltpu.VMEM((WB, D), jnp.float32),
            pltpu.SemaphoreType.DMA,
            pltpu.SemaphoreType.DMA,
            pltpu.VMEM_SHARED((NP, D), jnp.float32),
        ],
    )
    hop = pl.kernel(
        _hop,
        out_type=jax.ShapeDtypeStruct((NC, NP, D), jnp.float32),
        mesh=mesh,
        scratch_types=[
            pltpu.VMEM((C,), jnp.int32),
            pltpu.VMEM((C,), jnp.int32),
            pltpu.VMEM((C,), jnp.int32),
            pltpu.VMEM((C,), jnp.int32),
            pltpu.VMEM((WB, D), jnp.float32),
            pltpu.VMEM((WB, D), jnp.float32),
            pltpu.SemaphoreType.DMA,
            pltpu.SemaphoreType.DMA,
            pltpu.SemaphoreType.DMA,
            pltpu.SemaphoreType.DMA,
            pltpu.VMEM_SHARED((NP, D), jnp.float32),
        ],
    )
    return deg, hop


_BR = 2048


def _relu_scale(x, s):
    def body(x_ref, s_ref, o_ref):
        o_ref[...] = jnp.maximum(x_ref[...], 0.0) * s_ref[...]

    return pl.pallas_call(
        body,
        grid=(NP // _BR,),
        in_specs=[
            pl.BlockSpec((_BR, D), lambda i: (i, 0)),
            pl.BlockSpec((_BR, 1), lambda i: (i, 0)),
        ],
        out_specs=pl.BlockSpec((_BR, D), lambda i: (i, 0)),
        out_shape=jax.ShapeDtypeStruct((NP, D), jnp.float32),
    )(x, s)


def _combine(p0, p1, s):
    def body(a_ref, b_ref, s_ref, o_ref):
        o_ref[...] = (a_ref[...] + b_ref[...]) * s_ref[...]

    return pl.pallas_call(
        body,
        grid=(NP // _BR,),
        in_specs=[
            pl.BlockSpec((_BR, D), lambda i: (i, 0)),
            pl.BlockSpec((_BR, D), lambda i: (i, 0)),
            pl.BlockSpec((_BR, 1), lambda i: (i, 0)),
        ],
        out_specs=pl.BlockSpec((_BR, D), lambda i: (i, 0)),
        out_shape=jax.ShapeDtypeStruct((NP, D), jnp.float32),
    )(p0, p1, s)


def kernel(x, edge_index):
    pad = jnp.full((EP - E,), NP - 1, dtype=jnp.int32)
    src = jnp.concatenate([edge_index[0], pad]).reshape(NW, NCHUNK, C)
    dst = jnp.concatenate([edge_index[1], pad]).reshape(NW, NCHUNK, C)

    deg_call, hop_call = _sc_kernels()
    ones_rows = jnp.ones((C, D), jnp.float32)
    zrows = jnp.zeros((WB, D), jnp.float32)
    ps = deg_call(src, ones_rows, zrows)
    pd = deg_call(dst, ones_rows, zrows)
    deg_s = ps[0, :, 0] + ps[1, :, 0]
    deg_d = pd[0, :, 0] + pd[1, :, 0]
    s_src = jnp.where(deg_s > 0, lax.rsqrt(jnp.maximum(deg_s, 1.0)), 0.0)
    s_dst = jnp.where(deg_d > 0, lax.rsqrt(jnp.maximum(deg_d, 1.0)), 0.0)

    xp = jnp.pad(x, ((0, NP - N), (0, 0)))
    u = _relu_scale(xp, s_src[:, None])
    s_mid = (s_src * s_dst)[:, None]
    for t in range(K):
        p = hop_call(u, src, dst, zrows)
        u = _combine(p[0], p[1], s_mid if t < K - 1 else s_dst[:, None])
    return u[:N]
